# initial kernel scaffold (unmeasured)
import jax
import jax.numpy as jnp
from jax import lax
from jax.experimental import pallas as pl
from jax.experimental.pallas import tpu as pltpu

N_DEV = 32
SQ = 1024
SKV = 1024
HQ_LOCAL = 8
DH = 128
DMODEL = 1024
HEADS_COLS = HQ_LOCAL * DH
SCALE = 0.08838834764831843

DIST = (1, 2, 8, 4, 16)
RS_ROWS = (512, 256, 128, 64, 32)
RS_OFF = (0, 512, 768, 896, 960)
AG_OFF = {4: 1024, 3: 1056, 2: 1120, 1: 1248, 0: 1504}


def kernel(x, Wq, K_ext, V_ext, Wo):
    pos = lax.axis_index("i")
    Wq_s = lax.dynamic_slice_in_dim(Wq, pos * HEADS_COLS, HEADS_COLS, axis=1)
    Wo_s = lax.dynamic_slice_in_dim(Wo, pos * HEADS_COLS, HEADS_COLS, axis=0)

    def body(x_ref, wq_ref, k_ref, v_ref, wo_ref, out_ref,
             ctx_ref, acc_ref, comm_ref, rs_send, rs_recv, ag_send, ag_recv):
        my = lax.axis_index("i")

        xm = x_ref[0]
        q = jnp.dot(xm, wq_ref[...], preferred_element_type=jnp.float32)

        qi = lax.broadcasted_iota(jnp.int32, (SQ, SKV), 0)
        ki = lax.broadcasted_iota(jnp.int32, (SQ, SKV), 1)
        mask = (jnp.abs(qi - ki) <= 128) | (ki < 32) | (qi < 32)

        for h in range(HQ_LOCAL):
            qh = q[:, h * DH:(h + 1) * DH]
            kh = k_ref[0, :, h, :]
            s = lax.dot_general(
                qh, kh, (((1,), (1,)), ((), ())),
                preferred_element_type=jnp.float32,
            ) * SCALE
            s = jnp.where(mask, s, -1e9)
            m = jnp.max(s, axis=1, keepdims=True)
            e = jnp.exp(s - m)
            p = e / jnp.sum(e, axis=1, keepdims=True)
            ctx_ref[:, h * DH:(h + 1) * DH] = jnp.dot(
                p, v_ref[0, :, h, :], preferred_element_type=jnp.float32
            )

        acc_ref[...] = jnp.dot(
            ctx_ref[...], wo_ref[...], preferred_element_type=jnp.float32
        )

        base = jnp.int32(0)
        for s in range(5):
            d = DIST[s]
            half = RS_ROWS[s]
            bit = (my // d) % 2
            partner = my ^ d
            send_base = base + (1 - bit) * half
            keep_base = base + bit * half
            off = RS_OFF[s]
            rdma = pltpu.make_async_remote_copy(
                src_ref=acc_ref.at[pl.ds(send_base, half), :],
                dst_ref=comm_ref.at[pl.ds(off, half), :],
                send_sem=rs_send.at[s],
                recv_sem=rs_recv.at[s],
                device_id=(partner,),
                device_id_type=pl.DeviceIdType.MESH,
            )
            rdma.start()
            rdma.wait()
            acc_ref[pl.ds(keep_base, half), :] = (
                acc_ref[pl.ds(keep_base, half), :]
                + comm_ref[pl.ds(off, half), :]
            )
            base = keep_base

        for s in range(4, -1, -1):
            size = RS_ROWS[s]
            partner = my ^ DIST[s]
            pbase = base ^ size
            off = AG_OFF[s]
            rdma = pltpu.make_async_remote_copy(
                src_ref=acc_ref.at[pl.ds(base, size), :],
                dst_ref=comm_ref.at[pl.ds(off, size), :],
                send_sem=ag_send.at[s],
                recv_sem=ag_recv.at[s],
                device_id=(partner,),
                device_id_type=pl.DeviceIdType.MESH,
            )
            rdma.start()
            rdma.wait()
            acc_ref[pl.ds(pbase, size), :] = comm_ref[pl.ds(off, size), :]
            base = base & (~size)

        out_ref[0] = acc_ref[...]

    return pl.pallas_call(
        body,
        out_shape=jax.ShapeDtypeStruct((1, SQ, DMODEL), jnp.float32),
        in_specs=[pl.BlockSpec(memory_space=pltpu.VMEM)] * 5,
        out_specs=pl.BlockSpec(memory_space=pltpu.VMEM),
        scratch_shapes=[
            pltpu.VMEM((SQ, HEADS_COLS), jnp.float32),
            pltpu.VMEM((SQ, DMODEL), jnp.float32),
            pltpu.VMEM((2048, DMODEL), jnp.float32),
            pltpu.SemaphoreType.DMA((5,)),
            pltpu.SemaphoreType.DMA((5,)),
            pltpu.SemaphoreType.DMA((5,)),
            pltpu.SemaphoreType.DMA((5,)),
        ],
        compiler_params=pltpu.CompilerParams(collective_id=0),
    )(x, Wq_s, K_ext, V_ext, Wo_s)


# baseline (device time: 158219 ns/iter reference)
import jax
import jax.numpy as jnp
from jax import lax
from jax.experimental import pallas as pl
from jax.experimental.pallas import tpu as pltpu

N_DEV = 32
SQ = 1024
SKV = 1024
HQ_LOCAL = 8
DH = 128
DMODEL = 1024
HEADS_COLS = HQ_LOCAL * DH
SCALE = 0.08838834764831843

DIST = (1, 2, 8, 4, 16)
RS_ROWS = (512, 256, 128, 64, 32)
RS_OFF = (0, 512, 768, 896, 960)
AG_OFF = {4: 1024, 3: 1056, 2: 1120, 1: 1248, 0: 1504}


def kernel(x, Wq, K_ext, V_ext, Wo):
    pos = lax.axis_index("i")
    Wq_s = lax.dynamic_slice_in_dim(Wq, pos * HEADS_COLS, HEADS_COLS, axis=1)
    Wo_s = lax.dynamic_slice_in_dim(Wo, pos * HEADS_COLS, HEADS_COLS, axis=0)

    def body(x_ref, wq_ref, k_ref, v_ref, wo_ref, out_ref,
             ctx_ref, acc_ref, comm_ref, rs_send, rs_recv, ag_send, ag_recv):
        my = lax.axis_index("i")

        xm = x_ref[0]
        q = jnp.dot(xm, wq_ref[...], preferred_element_type=jnp.float32)

        qi = lax.broadcasted_iota(jnp.int32, (SQ, SKV), 0)
        ki = lax.broadcasted_iota(jnp.int32, (SQ, SKV), 1)
        mask = (jnp.abs(qi - ki) <= 128) | (ki < 32) | (qi < 32)

        for h in range(HQ_LOCAL):
            qh = q[:, h * DH:(h + 1) * DH]
            kh = k_ref[0, :, h, :]
            s = lax.dot_general(
                qh, kh, (((1,), (1,)), ((), ())),
                preferred_element_type=jnp.float32,
            ) * SCALE
            s = jnp.where(mask, s, -1e9)
            m = jnp.max(s, axis=1, keepdims=True)
            e = jnp.exp(s - m)
            p = e / jnp.sum(e, axis=1, keepdims=True)
            ctx_ref[:, h * DH:(h + 1) * DH] = jnp.dot(
                p, v_ref[0, :, h, :], preferred_element_type=jnp.float32
            )

        acc_ref[...] = jnp.dot(
            ctx_ref[...], wo_ref[...], preferred_element_type=jnp.float32
        )

        base = jnp.int32(0)
        for s in range(5):
            d = DIST[s]
            half = RS_ROWS[s]
            bit = (my // d) % 2
            partner = my ^ d
            send_base = pl.multiple_of(base + (1 - bit) * half, 32)
            keep_base = pl.multiple_of(base + bit * half, 32)
            off = RS_OFF[s]
            rdma = pltpu.make_async_remote_copy(
                src_ref=acc_ref.at[pl.ds(send_base, half), :],
                dst_ref=comm_ref.at[pl.ds(off, half), :],
                send_sem=rs_send.at[s],
                recv_sem=rs_recv.at[s],
                device_id=(partner,),
                device_id_type=pl.DeviceIdType.MESH,
            )
            rdma.start()
            rdma.wait()
            acc_ref[pl.ds(keep_base, half), :] = (
                acc_ref[pl.ds(keep_base, half), :]
                + comm_ref[pl.ds(off, half), :]
            )
            base = keep_base

        for s in range(4, -1, -1):
            size = RS_ROWS[s]
            partner = my ^ DIST[s]
            pbase = pl.multiple_of(base ^ size, 32)
            base = pl.multiple_of(base, 32)
            off = AG_OFF[s]
            rdma = pltpu.make_async_remote_copy(
                src_ref=acc_ref.at[pl.ds(base, size), :],
                dst_ref=comm_ref.at[pl.ds(off, size), :],
                send_sem=ag_send.at[s],
                recv_sem=ag_recv.at[s],
                device_id=(partner,),
                device_id_type=pl.DeviceIdType.MESH,
            )
            rdma.start()
            rdma.wait()
            acc_ref[pl.ds(pbase, size), :] = comm_ref[pl.ds(off, size), :]
            base = base & (~size)

        out_ref[0] = acc_ref[...]

    return pl.pallas_call(
        body,
        out_shape=jax.ShapeDtypeStruct((1, SQ, DMODEL), jnp.float32),
        in_specs=[pl.BlockSpec(memory_space=pltpu.VMEM)] * 5,
        out_specs=pl.BlockSpec(memory_space=pltpu.VMEM),
        scratch_shapes=[
            pltpu.VMEM((SQ, HEADS_COLS), jnp.float32),
            pltpu.VMEM((SQ, DMODEL), jnp.float32),
            pltpu.VMEM((2048, DMODEL), jnp.float32),
            pltpu.SemaphoreType.DMA((5,)),
            pltpu.SemaphoreType.DMA((5,)),
            pltpu.SemaphoreType.DMA((5,)),
            pltpu.SemaphoreType.DMA((5,)),
        ],
    )(x, Wq_s, K_ext, V_ext, Wo_s)
